# Initial kernel scaffold; baseline (speedup 1.0000x reference)
import jax
import jax.numpy as jnp
from jax.experimental import pallas as pl


def kernel(x, edge_index, batch, features, W_in, b_in, W_gcn, b_gcn, ln_g, ln_b, pool_p, Wf, bf, gf, betaf, W1, b1, g1, beta1, W2, b2, W3, b3):
    raise NotImplementedError("write your pallas kernel here")


# speedup vs baseline: 4.6708x; 4.6708x over previous
import functools

import jax
import jax.numpy as jnp
from jax import lax
from jax.experimental import pallas as pl
from jax.experimental.pallas import tpu as pltpu
from jax.experimental.pallas import tpu_sc as plsc

N = 10000
E = 320000
HID = 128
NL = 3
NG = 3

NPAD = 10240
NW = 32
CHUNK = 128
EPAD = 327680
NCHUNK = EPAD // NW // CHUNK
ROWS_PER_TILE = NPAD // 16

_KS = [N // 2, N // 4, N // 8]


def _gelu(t):
    return 0.5 * t * (1.0 + lax.erf(t * (2.0 ** -0.5)))


def _ln(t, g, b):
    mu = jnp.mean(t, axis=-1, keepdims=True)
    v = jnp.mean((t - mu) ** 2, axis=-1, keepdims=True)
    return (t - mu) * lax.rsqrt(v + 1e-5) * g + b


@functools.lru_cache(maxsize=None)
def _make_edge_accum(D):
    mesh = plsc.VectorSubcoreMesh(core_axis_name="c", subcore_axis_name="s")

    @functools.partial(
        pl.kernel,
        out_type=jax.ShapeDtypeStruct((2, NPAD, D), jnp.float32),
        mesh=mesh,
        scratch_types=[
            pltpu.VMEM((NCHUNK, CHUNK), jnp.int32),
            pltpu.VMEM((NCHUNK, CHUNK), jnp.int32),
            pltpu.VMEM((CHUNK, D), jnp.float32),
            pltpu.VMEM_SHARED((NPAD, D), jnp.float32),
            pltpu.SemaphoreType.DMA,
        ],
    )
    def edge_accum(src_hbm, dst_hbm, table_hbm, zeros_hbm, out_hbm,
                   sidx, didx, buf, acc, sem):
        c = lax.axis_index("c")
        s = lax.axis_index("s")
        wid = s * 2 + c
        pltpu.sync_copy(zeros_hbm.at[pl.ds(s * ROWS_PER_TILE, ROWS_PER_TILE)],
                        acc.at[pl.ds(s * ROWS_PER_TILE, ROWS_PER_TILE)])
        pltpu.sync_copy(src_hbm.at[wid], sidx)
        pltpu.sync_copy(dst_hbm.at[wid], didx)
        plsc.subcore_barrier()

        def body(j, carry):
            pltpu.async_copy(table_hbm.at[sidx.at[j]], buf, sem).wait()
            pltpu.sync_copy(buf, acc.at[didx.at[j]], add=True)
            return carry

        lax.fori_loop(0, NCHUNK, body, 0)
        plsc.subcore_barrier()
        pltpu.sync_copy(acc.at[pl.ds(s * ROWS_PER_TILE, ROWS_PER_TILE)],
                        out_hbm.at[c, pl.ds(s * ROWS_PER_TILE, ROWS_PER_TILE)])

    return edge_accum


def _edge_accum_128(*args):
    return _make_edge_accum(HID)(*args)


def _input_body(x_ref, w_ref, b_ref, alive_ref, h_ref, r0_ref):
    h = _gelu(jnp.dot(x_ref[...], w_ref[...],
                      preferred_element_type=jnp.float32) + b_ref[...])
    h_ref[...] = h
    r0 = jnp.sum(h * alive_ref[...], axis=0, keepdims=True) / N
    r0_ref[...] = jnp.broadcast_to(r0, (8, HID))


def _start_body(h_ref, w_ref, alive_ref, dg_ref, g_ref, dinv_ref):
    alive = alive_ref[...]
    deg = 1.0 + alive * (dg_ref[0][:, 0:1] + dg_ref[1][:, 0:1])
    dinv = lax.rsqrt(deg)
    dinv_ref[...] = dinv
    g_ref[...] = dinv * alive * jnp.dot(h_ref[...], w_ref[...],
                                        preferred_element_type=jnp.float32)


def _mid_body(g_ref, a_ref, dinv_ref, b_ref, w_ref, alive_ref, gout_ref):
    dinv = dinv_ref[...]
    h = _gelu(dinv * (a_ref[0] + a_ref[1] + g_ref[...]) + b_ref[...])
    gout_ref[...] = dinv * alive_ref[...] * jnp.dot(
        h, w_ref[...], preferred_element_type=jnp.float32)


def _pool_body(g_ref, a_ref, dinv_ref, b_ref, lng_ref, lnb_ref, p_ref,
               alive_ref, hout_ref, alive_out_ref, arow_ref, r_ref, *, k):
    dinv = dinv_ref[...]
    h = _gelu(dinv * (a_ref[0] + a_ref[1] + g_ref[...]) + b_ref[...])
    hn = _ln(h, lng_ref[...], lnb_ref[...])
    p = p_ref[...]
    pn = lax.rsqrt(jnp.sum(p * p))
    score = jnp.tanh(jnp.dot(hn, p, preferred_element_type=jnp.float32) * pn)
    alive = alive_ref[...]
    u = lax.bitcast_convert_type(score, jnp.uint32)
    ukey = jnp.where(u >= jnp.uint32(0x80000000), ~u,
                     u | jnp.uint32(0x80000000))
    ukey = jnp.where(alive > 0.0, ukey, jnp.uint32(0))

    def bit_step(i, t):
        cand = t | (jnp.uint32(1) << (jnp.uint32(31) - i.astype(jnp.uint32)))
        cnt = jnp.sum(jnp.where(ukey >= cand, 1.0, 0.0))
        return jnp.where(cnt >= k, cand, t)

    t = lax.fori_loop(0, 32, bit_step, jnp.uint32(0))
    keep = (ukey >= t).astype(jnp.float32)
    hout = hn * score * keep
    hout_ref[...] = hout
    alive_out_ref[...] = keep
    arow_ref[...] = jnp.broadcast_to(keep, (NPAD, HID))
    r = jnp.sum(hout, axis=0, keepdims=True) / k
    r_ref[...] = jnp.broadcast_to(r, (8, HID))


def _head_body(r0_ref, r1_ref, r2_ref, r3_ref, feat_ref, wf_ref, bf_ref,
               gf_ref, betaf_ref, w1_ref, b1_ref, g1_ref, beta1_ref,
               w2_ref, b2_ref, w3_ref, b3_ref, out_ref):
    f = jnp.dot(feat_ref[...], wf_ref[...],
                preferred_element_type=jnp.float32) + bf_ref[...]
    f = _gelu(_ln(f, gf_ref[...], betaf_ref[...]))
    w1 = w1_ref[...]
    c = (jnp.dot(r0_ref[...], w1[0:128], preferred_element_type=jnp.float32)
         + jnp.dot(r1_ref[...], w1[128:256], preferred_element_type=jnp.float32)
         + jnp.dot(r2_ref[...], w1[256:384], preferred_element_type=jnp.float32)
         + jnp.dot(r3_ref[...], w1[384:512], preferred_element_type=jnp.float32)
         + jnp.dot(f, w1[512:640], preferred_element_type=jnp.float32)
         + b1_ref[...])
    o = _gelu(_ln(c, g1_ref[...], beta1_ref[...]))
    o = _gelu(jnp.dot(o, w2_ref[...], preferred_element_type=jnp.float32)
              + b2_ref[...])
    out_ref[...] = jnp.dot(o, w3_ref[...],
                           preferred_element_type=jnp.float32) + b3_ref[...]


def _tc_call(body, out_shapes):
    return pl.pallas_call(body, out_shape=out_shapes)


_NODE_F = jax.ShapeDtypeStruct((NPAD, HID), jnp.float32)
_NODE_1 = jax.ShapeDtypeStruct((NPAD, 1), jnp.float32)
_ROW8 = jax.ShapeDtypeStruct((8, HID), jnp.float32)


def kernel(x, edge_index, batch, features, W_in, b_in, W_gcn, b_gcn, ln_g,
           ln_b, pool_p, Wf, bf, gf, betaf, W1, b1, g1, beta1, W2, b2, W3,
           b3):
    f32 = jnp.float32
    xp = jnp.zeros((NPAD, HID), f32).at[:N].set(x)
    alive = (jnp.arange(NPAD, dtype=jnp.int32) < N).astype(f32)[:, None]
    arow = jnp.broadcast_to(alive, (NPAD, HID))
    srcp = jnp.full((EPAD,), N, jnp.int32).at[:E].set(edge_index[0])
    dstp = jnp.full((EPAD,), N, jnp.int32).at[:E].set(edge_index[1])
    srcp = srcp.reshape(NW, NCHUNK, CHUNK)
    dstp = dstp.reshape(NW, NCHUNK, CHUNK)
    zeros128 = jnp.zeros((NPAD, HID), f32)

    h, r0 = _tc_call(_input_body, (_NODE_F, _ROW8))(
        xp, W_in, b_in.reshape(1, HID), alive)
    reprs = [r0]

    for i in range(NL):
        dg = _edge_accum_128(srcp, dstp, arow, zeros128)
        g, dinv = _tc_call(_start_body, (_NODE_F, _NODE_1))(
            h, W_gcn[i, 0], alive, dg)
        for j in range(NG):
            a = _edge_accum_128(srcp, dstp, g, zeros128)
            if j < NG - 1:
                (g,) = _tc_call(_mid_body, (_NODE_F,))(
                    g, a, dinv, b_gcn[i, j].reshape(1, HID), W_gcn[i, j + 1],
                    alive)
            else:
                h, alive, arow, r = _tc_call(
                    functools.partial(_pool_body, k=_KS[i]),
                    (_NODE_F, _NODE_1, _NODE_F, _ROW8))(
                        g, a, dinv, b_gcn[i, j].reshape(1, HID),
                        ln_g[i].reshape(1, HID), ln_b[i].reshape(1, HID),
                        pool_p[i].reshape(HID, 1), alive)
                reprs.append(r)

    featp = jnp.zeros((8, HID), f32).at[0, : features.shape[1]].set(features[0])
    wfp = jnp.zeros((HID, HID), f32).at[: Wf.shape[0]].set(Wf)
    w2p = jnp.zeros((HID, HID), f32).at[:, : W2.shape[1]].set(W2)
    b2p = jnp.zeros((1, HID), f32).at[0, : W2.shape[1]].set(b2)
    w3p = jnp.zeros((HID, HID), f32).at[: W3.shape[0], 0].set(W3[:, 0])
    b3p = jnp.zeros((1, HID), f32).at[0, 0].set(b3[0])

    out8 = _tc_call(_head_body, (_ROW8,))(
        reprs[0], reprs[1], reprs[2], reprs[3], featp, wfp,
        bf.reshape(1, HID), gf.reshape(1, HID), betaf.reshape(1, HID),
        W1, b1.reshape(1, HID), g1.reshape(1, HID), beta1.reshape(1, HID),
        w2p, b2p, w3p, b3p)[0]
    return out8[0, 0:1]


# speedup vs baseline: 5.2406x; 1.1220x over previous
import functools

import jax
import jax.numpy as jnp
from jax import lax
from jax.experimental import pallas as pl
from jax.experimental.pallas import tpu as pltpu
from jax.experimental.pallas import tpu_sc as plsc

N = 10000
E = 320000
HID = 128
NL = 3
NG = 3

NPAD = 10240
NW = 32
CHUNK = 128
HALF = CHUNK // 2
EPAD = 327680
NCHUNK = EPAD // NW // CHUNK
ROWS_PER_TILE = NPAD // 16

_KS = [N // 2, N // 4, N // 8]


def _gelu(t):
    return 0.5 * t * (1.0 + lax.erf(t * (2.0 ** -0.5)))


def _ln(t, g, b):
    mu = jnp.mean(t, axis=-1, keepdims=True)
    v = jnp.mean((t - mu) ** 2, axis=-1, keepdims=True)
    return (t - mu) * lax.rsqrt(v + 1e-5) * g + b


@functools.lru_cache(maxsize=None)
def _make_edge_accum(D):
    mesh = plsc.VectorSubcoreMesh(core_axis_name="c", subcore_axis_name="s")

    @functools.partial(
        pl.kernel,
        out_type=jax.ShapeDtypeStruct((2, NPAD, D), jnp.float32),
        mesh=mesh,
        scratch_types=[
            pltpu.VMEM((NCHUNK, CHUNK), jnp.int32),
            pltpu.VMEM((NCHUNK, CHUNK), jnp.int32),
            pltpu.VMEM((HALF, D), jnp.float32),
            pltpu.VMEM((HALF, D), jnp.float32),
            pltpu.VMEM_SHARED((NPAD, D), jnp.float32),
            pltpu.SemaphoreType.DMA,
            pltpu.SemaphoreType.DMA,
        ],
    )
    def edge_accum(src_hbm, dst_hbm, table_hbm, zeros_hbm, out_hbm,
                   sidx, didx, buf0, buf1, acc, sem0, sem1):
        c = lax.axis_index("c")
        s = lax.axis_index("s")
        wid = s * 2 + c
        pltpu.sync_copy(zeros_hbm.at[pl.ds(s * ROWS_PER_TILE, ROWS_PER_TILE)],
                        acc.at[pl.ds(s * ROWS_PER_TILE, ROWS_PER_TILE)])
        pltpu.sync_copy(src_hbm.at[wid], sidx)
        pltpu.sync_copy(dst_hbm.at[wid], didx)
        plsc.subcore_barrier()

        pltpu.async_copy(table_hbm.at[sidx.at[0, pl.ds(0, HALF)]], buf0, sem0)

        def body(j, carry):
            c1 = pltpu.async_copy(
                table_hbm.at[sidx.at[j, pl.ds(HALF, HALF)]], buf1, sem1)
            pltpu.make_async_copy(
                table_hbm.at[sidx.at[j, pl.ds(0, HALF)]], buf0, sem0).wait()
            pltpu.sync_copy(buf0, acc.at[didx.at[j, pl.ds(0, HALF)]], add=True)

            @pl.when(j + 1 < NCHUNK)
            def _():
                pltpu.async_copy(
                    table_hbm.at[sidx.at[j + 1, pl.ds(0, HALF)]], buf0, sem0)

            c1.wait()
            pltpu.sync_copy(buf1, acc.at[didx.at[j, pl.ds(HALF, HALF)]],
                            add=True)
            return carry

        lax.fori_loop(0, NCHUNK, body, 0)
        plsc.subcore_barrier()
        pltpu.sync_copy(acc.at[pl.ds(s * ROWS_PER_TILE, ROWS_PER_TILE)],
                        out_hbm.at[c, pl.ds(s * ROWS_PER_TILE, ROWS_PER_TILE)])

    return edge_accum


def _edge_accum_128(*args):
    return _make_edge_accum(HID)(*args)


def _input_body(x_ref, w_ref, b_ref, alive_ref, h_ref, r0_ref):
    h = _gelu(jnp.dot(x_ref[...], w_ref[...],
                      preferred_element_type=jnp.float32) + b_ref[...])
    h_ref[...] = h
    r0 = jnp.sum(h * alive_ref[...], axis=0, keepdims=True) / N
    r0_ref[...] = jnp.broadcast_to(r0, (8, HID))


def _start_body(h_ref, w_ref, alive_ref, dg_ref, g_ref, dinv_ref):
    alive = alive_ref[...]
    deg = 1.0 + alive * (dg_ref[0][:, 0:1] + dg_ref[1][:, 0:1])
    dinv = lax.rsqrt(deg)
    dinv_ref[...] = dinv
    g_ref[...] = dinv * alive * jnp.dot(h_ref[...], w_ref[...],
                                        preferred_element_type=jnp.float32)


def _mid_body(g_ref, a_ref, dinv_ref, b_ref, w_ref, alive_ref, gout_ref):
    dinv = dinv_ref[...]
    h = _gelu(dinv * (a_ref[0] + a_ref[1] + g_ref[...]) + b_ref[...])
    gout_ref[...] = dinv * alive_ref[...] * jnp.dot(
        h, w_ref[...], preferred_element_type=jnp.float32)


def _pool_body(g_ref, a_ref, dinv_ref, b_ref, lng_ref, lnb_ref, p_ref,
               alive_ref, hout_ref, alive_out_ref, arow_ref, r_ref, *, k):
    dinv = dinv_ref[...]
    h = _gelu(dinv * (a_ref[0] + a_ref[1] + g_ref[...]) + b_ref[...])
    hn = _ln(h, lng_ref[...], lnb_ref[...])
    p = p_ref[...]
    pn = lax.rsqrt(jnp.sum(p * p))
    score = jnp.tanh(jnp.dot(hn, p, preferred_element_type=jnp.float32) * pn)
    alive = alive_ref[...]
    u = lax.bitcast_convert_type(score, jnp.uint32)
    ukey = jnp.where(u >= jnp.uint32(0x80000000), ~u,
                     u | jnp.uint32(0x80000000))
    ukey = jnp.where(alive > 0.0, ukey, jnp.uint32(0))

    def bit_step(i, t):
        cand = t | (jnp.uint32(1) << (jnp.uint32(31) - i.astype(jnp.uint32)))
        cnt = jnp.sum(jnp.where(ukey >= cand, 1.0, 0.0))
        return jnp.where(cnt >= k, cand, t)

    t = lax.fori_loop(0, 32, bit_step, jnp.uint32(0))
    keep = (ukey >= t).astype(jnp.float32)
    hout = hn * score * keep
    hout_ref[...] = hout
    alive_out_ref[...] = keep
    arow_ref[...] = jnp.broadcast_to(keep, (NPAD, HID))
    r = jnp.sum(hout, axis=0, keepdims=True) / k
    r_ref[...] = jnp.broadcast_to(r, (8, HID))


def _head_body(r0_ref, r1_ref, r2_ref, r3_ref, feat_ref, wf_ref, bf_ref,
               gf_ref, betaf_ref, w1_ref, b1_ref, g1_ref, beta1_ref,
               w2_ref, b2_ref, w3_ref, b3_ref, out_ref):
    f = jnp.dot(feat_ref[...], wf_ref[...],
                preferred_element_type=jnp.float32) + bf_ref[...]
    f = _gelu(_ln(f, gf_ref[...], betaf_ref[...]))
    w1 = w1_ref[...]
    c = (jnp.dot(r0_ref[...], w1[0:128], preferred_element_type=jnp.float32)
         + jnp.dot(r1_ref[...], w1[128:256], preferred_element_type=jnp.float32)
         + jnp.dot(r2_ref[...], w1[256:384], preferred_element_type=jnp.float32)
         + jnp.dot(r3_ref[...], w1[384:512], preferred_element_type=jnp.float32)
         + jnp.dot(f, w1[512:640], preferred_element_type=jnp.float32)
         + b1_ref[...])
    o = _gelu(_ln(c, g1_ref[...], beta1_ref[...]))
    o = _gelu(jnp.dot(o, w2_ref[...], preferred_element_type=jnp.float32)
              + b2_ref[...])
    out_ref[...] = jnp.dot(o, w3_ref[...],
                           preferred_element_type=jnp.float32) + b3_ref[...]


def _tc_call(body, out_shapes):
    return pl.pallas_call(body, out_shape=out_shapes)


_NODE_F = jax.ShapeDtypeStruct((NPAD, HID), jnp.float32)
_NODE_1 = jax.ShapeDtypeStruct((NPAD, 1), jnp.float32)
_ROW8 = jax.ShapeDtypeStruct((8, HID), jnp.float32)


def kernel(x, edge_index, batch, features, W_in, b_in, W_gcn, b_gcn, ln_g,
           ln_b, pool_p, Wf, bf, gf, betaf, W1, b1, g1, beta1, W2, b2, W3,
           b3):
    f32 = jnp.float32
    xp = jnp.zeros((NPAD, HID), f32).at[:N].set(x)
    alive = (jnp.arange(NPAD, dtype=jnp.int32) < N).astype(f32)[:, None]
    arow = jnp.broadcast_to(alive, (NPAD, HID))
    srcp = jnp.full((EPAD,), N, jnp.int32).at[:E].set(edge_index[0])
    dstp = jnp.full((EPAD,), N, jnp.int32).at[:E].set(edge_index[1])
    srcp = srcp.reshape(NW, NCHUNK, CHUNK)
    dstp = dstp.reshape(NW, NCHUNK, CHUNK)
    zeros128 = jnp.zeros((NPAD, HID), f32)

    h, r0 = _tc_call(_input_body, (_NODE_F, _ROW8))(
        xp, W_in, b_in.reshape(1, HID), alive)
    reprs = [r0]

    for i in range(NL):
        dg = _edge_accum_128(srcp, dstp, arow, zeros128)
        g, dinv = _tc_call(_start_body, (_NODE_F, _NODE_1))(
            h, W_gcn[i, 0], alive, dg)
        for j in range(NG):
            a = _edge_accum_128(srcp, dstp, g, zeros128)
            if j < NG - 1:
                (g,) = _tc_call(_mid_body, (_NODE_F,))(
                    g, a, dinv, b_gcn[i, j].reshape(1, HID), W_gcn[i, j + 1],
                    alive)
            else:
                h, alive, arow, r = _tc_call(
                    functools.partial(_pool_body, k=_KS[i]),
                    (_NODE_F, _NODE_1, _NODE_F, _ROW8))(
                        g, a, dinv, b_gcn[i, j].reshape(1, HID),
                        ln_g[i].reshape(1, HID), ln_b[i].reshape(1, HID),
                        pool_p[i].reshape(HID, 1), alive)
                reprs.append(r)

    featp = jnp.zeros((8, HID), f32).at[0, : features.shape[1]].set(features[0])
    wfp = jnp.zeros((HID, HID), f32).at[: Wf.shape[0]].set(Wf)
    w2p = jnp.zeros((HID, HID), f32).at[:, : W2.shape[1]].set(W2)
    b2p = jnp.zeros((1, HID), f32).at[0, : W2.shape[1]].set(b2)
    w3p = jnp.zeros((HID, HID), f32).at[: W3.shape[0], 0].set(W3[:, 0])
    b3p = jnp.zeros((1, HID), f32).at[0, 0].set(b3[0])

    out8 = _tc_call(_head_body, (_ROW8,))(
        reprs[0], reprs[1], reprs[2], reprs[3], featp, wfp,
        bf.reshape(1, HID), gf.reshape(1, HID), betaf.reshape(1, HID),
        W1, b1.reshape(1, HID), g1.reshape(1, HID), beta1.reshape(1, HID),
        w2p, b2p, w3p, b3p)[0]
    return out8[0, 0:1]


# speedup vs baseline: 5.4037x; 1.0311x over previous
import functools

import jax
import jax.numpy as jnp
from jax import lax
from jax.experimental import pallas as pl
from jax.experimental.pallas import tpu as pltpu
from jax.experimental.pallas import tpu_sc as plsc

N = 10000
E = 320000
HID = 128
NL = 3
NG = 3

NPAD = 10240
NW = 32
CHUNK = 128
HALF = CHUNK // 2
EPAD = 327680
NCHUNK = EPAD // NW // CHUNK
ROWS_PER_TILE = NPAD // 16

_KS = [N // 2, N // 4, N // 8]


def _gelu(t):
    return 0.5 * t * (1.0 + lax.erf(t * (2.0 ** -0.5)))


def _ln(t, g, b):
    mu = jnp.mean(t, axis=-1, keepdims=True)
    v = jnp.mean((t - mu) ** 2, axis=-1, keepdims=True)
    return (t - mu) * lax.rsqrt(v + 1e-5) * g + b


@functools.lru_cache(maxsize=None)
def _make_edge_accum(D):
    mesh = plsc.VectorSubcoreMesh(core_axis_name="c", subcore_axis_name="s")

    @functools.partial(
        pl.kernel,
        out_type=jax.ShapeDtypeStruct((2, NPAD, D), jnp.float32),
        mesh=mesh,
        scratch_types=[
            pltpu.VMEM((NCHUNK, CHUNK), jnp.int32),
            pltpu.VMEM((NCHUNK, CHUNK), jnp.int32),
            pltpu.VMEM((HALF, D), jnp.float32),
            pltpu.VMEM((HALF, D), jnp.float32),
            pltpu.VMEM_SHARED((NPAD, D), jnp.float32),
            pltpu.SemaphoreType.DMA,
            pltpu.SemaphoreType.DMA,
        ],
    )
    def edge_accum(src_hbm, dst_hbm, table_hbm, zeros_hbm, out_hbm,
                   sidx, didx, buf0, buf1, acc, sem0, sem1):
        c = lax.axis_index("c")
        s = lax.axis_index("s")
        wid = s * 2 + c
        pltpu.sync_copy(zeros_hbm, buf0)
        for t in range(ROWS_PER_TILE // HALF):
            pltpu.sync_copy(
                buf0, acc.at[pl.ds(s * ROWS_PER_TILE + t * HALF, HALF)])
        pltpu.sync_copy(src_hbm.at[wid], sidx)
        pltpu.sync_copy(dst_hbm.at[wid], didx)
        plsc.subcore_barrier()

        pltpu.async_copy(table_hbm.at[sidx.at[0, pl.ds(0, HALF)]], buf0, sem0)

        def body(j, carry):
            c1 = pltpu.async_copy(
                table_hbm.at[sidx.at[j, pl.ds(HALF, HALF)]], buf1, sem1)
            pltpu.make_async_copy(
                table_hbm.at[sidx.at[j, pl.ds(0, HALF)]], buf0, sem0).wait()
            pltpu.sync_copy(buf0, acc.at[didx.at[j, pl.ds(0, HALF)]], add=True)

            @pl.when(j + 1 < NCHUNK)
            def _():
                pltpu.async_copy(
                    table_hbm.at[sidx.at[j + 1, pl.ds(0, HALF)]], buf0, sem0)

            c1.wait()
            pltpu.sync_copy(buf1, acc.at[didx.at[j, pl.ds(HALF, HALF)]],
                            add=True)
            return carry

        lax.fori_loop(0, NCHUNK, body, 0)
        plsc.subcore_barrier()
        pltpu.sync_copy(acc.at[pl.ds(s * ROWS_PER_TILE, ROWS_PER_TILE)],
                        out_hbm.at[c, pl.ds(s * ROWS_PER_TILE, ROWS_PER_TILE)])

    return edge_accum


def _edge_accum_128(*args):
    return _make_edge_accum(HID)(*args)


def _input_body(x_ref, w_ref, b_ref, alive_ref, h_ref, r0_ref):
    h = _gelu(jnp.dot(x_ref[...], w_ref[...],
                      preferred_element_type=jnp.float32) + b_ref[...])
    h_ref[...] = h
    r0 = jnp.sum(h * alive_ref[...], axis=0, keepdims=True) / N
    r0_ref[...] = jnp.broadcast_to(r0, (8, HID))


def _start_body(h_ref, w_ref, alive_ref, dg_ref, g_ref, dinv_ref):
    alive = alive_ref[...]
    deg = 1.0 + alive * (dg_ref[0][:, 0:1] + dg_ref[1][:, 0:1])
    dinv = lax.rsqrt(deg)
    dinv_ref[...] = dinv
    g_ref[...] = dinv * alive * jnp.dot(h_ref[...], w_ref[...],
                                        preferred_element_type=jnp.float32)


def _mid_body(g_ref, a_ref, dinv_ref, b_ref, w_ref, alive_ref, gout_ref):
    dinv = dinv_ref[...]
    h = _gelu(dinv * (a_ref[0] + a_ref[1] + g_ref[...]) + b_ref[...])
    gout_ref[...] = dinv * alive_ref[...] * jnp.dot(
        h, w_ref[...], preferred_element_type=jnp.float32)


def _pool_body(g_ref, a_ref, dinv_ref, b_ref, lng_ref, lnb_ref, p_ref,
               alive_ref, hout_ref, alive_out_ref, arow_ref, r_ref, *, k):
    dinv = dinv_ref[...]
    h = _gelu(dinv * (a_ref[0] + a_ref[1] + g_ref[...]) + b_ref[...])
    hn = _ln(h, lng_ref[...], lnb_ref[...])
    p = p_ref[...]
    pn = lax.rsqrt(jnp.sum(p * p))
    score = jnp.tanh(jnp.dot(hn, p, preferred_element_type=jnp.float32) * pn)
    alive = alive_ref[...]
    u = lax.bitcast_convert_type(score, jnp.uint32)
    ukey = jnp.where(u >= jnp.uint32(0x80000000), ~u,
                     u | jnp.uint32(0x80000000))
    ukey = jnp.where(alive > 0.0, ukey, jnp.uint32(0))

    def bit_step(i, t):
        cand = t | (jnp.uint32(1) << (jnp.uint32(31) - i.astype(jnp.uint32)))
        cnt = jnp.sum(jnp.where(ukey >= cand, 1.0, 0.0))
        return jnp.where(cnt >= k, cand, t)

    t = lax.fori_loop(0, 32, bit_step, jnp.uint32(0))
    keep = (ukey >= t).astype(jnp.float32)
    hout = hn * score * keep
    hout_ref[...] = hout
    alive_out_ref[...] = keep
    arow_ref[...] = jnp.broadcast_to(keep, (NPAD, HID))
    r = jnp.sum(hout, axis=0, keepdims=True) / k
    r_ref[...] = jnp.broadcast_to(r, (8, HID))


def _head_body(r0_ref, r1_ref, r2_ref, r3_ref, feat_ref, wf_ref, bf_ref,
               gf_ref, betaf_ref, w1_ref, b1_ref, g1_ref, beta1_ref,
               w2_ref, b2_ref, w3_ref, b3_ref, out_ref):
    f = jnp.dot(feat_ref[...], wf_ref[...],
                preferred_element_type=jnp.float32) + bf_ref[...]
    f = _gelu(_ln(f, gf_ref[...], betaf_ref[...]))
    w1 = w1_ref[...]
    c = (jnp.dot(r0_ref[...], w1[0:128], preferred_element_type=jnp.float32)
         + jnp.dot(r1_ref[...], w1[128:256], preferred_element_type=jnp.float32)
         + jnp.dot(r2_ref[...], w1[256:384], preferred_element_type=jnp.float32)
         + jnp.dot(r3_ref[...], w1[384:512], preferred_element_type=jnp.float32)
         + jnp.dot(f, w1[512:640], preferred_element_type=jnp.float32)
         + b1_ref[...])
    o = _gelu(_ln(c, g1_ref[...], beta1_ref[...]))
    o = _gelu(jnp.dot(o, w2_ref[...], preferred_element_type=jnp.float32)
              + b2_ref[...])
    out_ref[...] = jnp.dot(o, w3_ref[...],
                           preferred_element_type=jnp.float32) + b3_ref[...]


def _tc_call(body, out_shapes):
    return pl.pallas_call(body, out_shape=out_shapes)


_NODE_F = jax.ShapeDtypeStruct((NPAD, HID), jnp.float32)
_NODE_1 = jax.ShapeDtypeStruct((NPAD, 1), jnp.float32)
_ROW8 = jax.ShapeDtypeStruct((8, HID), jnp.float32)


def kernel(x, edge_index, batch, features, W_in, b_in, W_gcn, b_gcn, ln_g,
           ln_b, pool_p, Wf, bf, gf, betaf, W1, b1, g1, beta1, W2, b2, W3,
           b3):
    f32 = jnp.float32
    xp = jnp.zeros((NPAD, HID), f32).at[:N].set(x)
    alive = (jnp.arange(NPAD, dtype=jnp.int32) < N).astype(f32)[:, None]
    arow = jnp.broadcast_to(alive, (NPAD, HID))
    srcp = jnp.full((EPAD,), N, jnp.int32).at[:E].set(edge_index[0])
    dstp = jnp.full((EPAD,), N, jnp.int32).at[:E].set(edge_index[1])
    srcp = srcp.reshape(NW, NCHUNK, CHUNK)
    dstp = dstp.reshape(NW, NCHUNK, CHUNK)
    zeros128 = jnp.zeros((HALF, HID), f32)

    h, r0 = _tc_call(_input_body, (_NODE_F, _ROW8))(
        xp, W_in, b_in.reshape(1, HID), alive)
    reprs = [r0]

    for i in range(NL):
        dg = _edge_accum_128(srcp, dstp, arow, zeros128)
        g, dinv = _tc_call(_start_body, (_NODE_F, _NODE_1))(
            h, W_gcn[i, 0], alive, dg)
        for j in range(NG):
            a = _edge_accum_128(srcp, dstp, g, zeros128)
            if j < NG - 1:
                (g,) = _tc_call(_mid_body, (_NODE_F,))(
                    g, a, dinv, b_gcn[i, j].reshape(1, HID), W_gcn[i, j + 1],
                    alive)
            else:
                h, alive, arow, r = _tc_call(
                    functools.partial(_pool_body, k=_KS[i]),
                    (_NODE_F, _NODE_1, _NODE_F, _ROW8))(
                        g, a, dinv, b_gcn[i, j].reshape(1, HID),
                        ln_g[i].reshape(1, HID), ln_b[i].reshape(1, HID),
                        pool_p[i].reshape(HID, 1), alive)
                reprs.append(r)

    featp = jnp.zeros((8, HID), f32).at[0, : features.shape[1]].set(features[0])
    wfp = jnp.zeros((HID, HID), f32).at[: Wf.shape[0]].set(Wf)
    w2p = jnp.zeros((HID, HID), f32).at[:, : W2.shape[1]].set(W2)
    b2p = jnp.zeros((1, HID), f32).at[0, : W2.shape[1]].set(b2)
    w3p = jnp.zeros((HID, HID), f32).at[: W3.shape[0], 0].set(W3[:, 0])
    b3p = jnp.zeros((1, HID), f32).at[0, 0].set(b3[0])

    out8 = _tc_call(_head_body, (_ROW8,))(
        reprs[0], reprs[1], reprs[2], reprs[3], featp, wfp,
        bf.reshape(1, HID), gf.reshape(1, HID), betaf.reshape(1, HID),
        W1, b1.reshape(1, HID), g1.reshape(1, HID), beta1.reshape(1, HID),
        w2p, b2p, w3p, b3p)[0]
    return out8[0, 0:1]
